# Initial kernel scaffold; baseline (speedup 1.0000x reference)
import jax
import jax.numpy as jnp
from jax.experimental import pallas as pl


def kernel(rel, edge_index, pattern, W_attn):
    raise NotImplementedError("write your pallas kernel here")


# speedup vs baseline: 11.2665x; 11.2665x over previous
import functools

import jax
import jax.numpy as jnp
from jax import lax
from jax.experimental import pallas as pl
from jax.experimental.pallas import tpu as pltpu
from jax.experimental.pallas import tpu_sc as plsc

N = 10000
E = 320000
D = 128
DW = 8
NC = 2
NS = 16
NW = NC * NS
EPW = E // NW
B = 80
NBLK = EPW // B
G = B // 16


def _sc_mesh():
    return plsc.VectorSubcoreMesh(core_axis_name="c", subcore_axis_name="s")


@functools.partial(
    pl.kernel,
    out_type=(jax.ShapeDtypeStruct((NC * N, D), jnp.float32),
              jax.ShapeDtypeStruct((NC * N, DW), jnp.float32)),
    mesh=_sc_mesh(),
    compiler_params=pltpu.CompilerParams(needs_layout_passes=False,
                                         use_tc_tiling_on_sc=False),
    scratch_types=[
        pltpu.VMEM((D,), jnp.float32),
        pltpu.VMEM((N,), jnp.float32),
        pltpu.VMEM((B,), jnp.int32),
        pltpu.VMEM((B,), jnp.int32),
        pltpu.VMEM((B, D), jnp.float32),
        pltpu.VMEM((B, D), jnp.float32),
        pltpu.VMEM((B, D), jnp.float32),
        pltpu.VMEM((B, DW), jnp.float32),
        pltpu.VMEM_SHARED((N, D), jnp.float32),
        pltpu.VMEM_SHARED((N, DW), jnp.float32),
        pltpu.SemaphoreType.DMA,
    ],
)
def _sc_main(rel_hbm, src_hbm, dst_hbm, pat_hbm, w1_hbm, b_hbm,
             outh_hbm, outd_hbm,
             w1_v, btab_v, src_v, dst_v, relrows_v, pat_v, contrib_v, pstg_v,
             hacc_sh, den_sh, gsem):
    cid = lax.axis_index("c")
    sid = lax.axis_index("s")
    wid = sid * NC + cid

    pltpu.sync_copy(w1_hbm, w1_v)
    pltpu.sync_copy(b_hbm, btab_v)

    zero16 = jnp.zeros((16,), jnp.float32)
    lane = lax.broadcasted_iota(jnp.int32, (16,), 0)

    def _zrow(r, carry):
        for k in range(D // 16):
            contrib_v[r, pl.ds(k * 16, 16)] = zero16
        return carry

    lax.fori_loop(0, B, _zrow, 0)
    for g in range(G):
        for c in range(DW):
            plsc.store_scatter(pstg_v, [g * 16 + lane,
                                        jnp.full((16,), c, jnp.int32)],
                               zero16)
    nchunk = N // B
    for i in range((nchunk + NS - 1) // NS):
        t = i * NS + sid

        @pl.when(t < nchunk)
        def _():
            pltpu.sync_copy(contrib_v, hacc_sh.at[pl.ds(t * B, B)])
            pltpu.sync_copy(pstg_v, den_sh.at[pl.ds(t * B, B)])

    plsc.subcore_barrier()

    w1c = [w1_v[pl.ds(k * 16, 16)] for k in range(D // 16)]
    zcol = jnp.zeros((16,), jnp.int32)

    def _block(blk, carry):
        base = wid * EPW + blk * B
        pltpu.sync_copy(src_hbm.at[pl.ds(base, B)], src_v)
        pltpu.sync_copy(dst_hbm.at[pl.ds(base, B)], dst_v)
        pltpu.async_copy(rel_hbm.at[src_v], relrows_v, gsem).wait()
        pltpu.sync_copy(pat_hbm.at[pl.ds(base, B)], pat_v)

        def _group(g, gcarry):
            dst16 = dst_v[pl.ds(g * 16, 16)]
            b16 = plsc.load_gather(btab_v, [dst16])
            pvec = zero16
            for j in range(16):
                e = g * 16 + j
                acc = zero16
                zs = []
                for k in range(D // 16):
                    r = relrows_v[e, pl.ds(k * 16, 16)]
                    pt = pat_v[e, pl.ds(k * 16, 16)]
                    z = r * pt
                    zs.append(z)
                    acc = acc + z * w1c[k]
                acc = acc + jnp.where(lane == j, b16, zero16)
                s = jnp.sum(acc)
                sv = lax.broadcast_in_dim(s, (16,), ())
                sv = jnp.where(sv >= 0, sv, sv * jnp.float32(0.01))
                p = jnp.exp(sv)
                for k in range(D // 16):
                    contrib_v[e, pl.ds(k * 16, 16)] = zs[k] * p
                pvec = jnp.where(lane == j, p, pvec)
            plsc.store_scatter(pstg_v, [g * 16 + lane, zcol], pvec)
            return gcarry

        lax.fori_loop(0, G, _group, 0)
        pltpu.sync_copy(contrib_v, hacc_sh.at[dst_v], add=True)
        pltpu.sync_copy(pstg_v, den_sh.at[dst_v], add=True)
        return carry

    lax.fori_loop(0, NBLK, _block, 0)
    plsc.subcore_barrier()
    for i in range((nchunk + NS - 1) // NS):
        t = i * NS + sid

        @pl.when(t < nchunk)
        def _():
            pltpu.sync_copy(hacc_sh.at[pl.ds(t * B, B)],
                            outh_hbm.at[pl.ds(cid * N + t * B, B)])
            pltpu.sync_copy(den_sh.at[pl.ds(t * B, B)],
                            outd_hbm.at[pl.ds(cid * N + t * B, B)])


def _pre_body(rel_ref, w2_ref, b_ref):
    b_ref[...] = jnp.sum(rel_ref[...] * w2_ref[...], axis=1, keepdims=True)


def _post_body(acc_ref, den_ref, o_ref):
    hs = acc_ref[0] + acc_ref[1]
    den = den_ref[0, :, 0:1] + den_ref[1, :, 0:1]
    o_ref[...] = jnp.where(den > 0, hs / den, jnp.float32(0.0))


@jax.jit
def kernel(rel, edge_index, pattern, W_attn):
    w1 = W_attn[0, :D]
    w2 = W_attn[0, D:].reshape(1, D)
    src = edge_index[0]
    dst = edge_index[1]

    b = pl.pallas_call(
        _pre_body,
        out_shape=jax.ShapeDtypeStruct((N, 1), jnp.float32),
    )(rel, w2)[:, 0]

    acc, den = _sc_main(rel, src, dst, pattern, w1, b)
    acc = acc.reshape(NC, N, D)
    den = den.reshape(NC, N, DW)

    h = pl.pallas_call(
        _post_body,
        out_shape=jax.ShapeDtypeStruct((N, D), jnp.float32),
    )(acc, den)
    return h


# speedup vs baseline: 14.4012x; 1.2782x over previous
import functools

import jax
import jax.numpy as jnp
from jax import lax
from jax.experimental import pallas as pl
from jax.experimental.pallas import tpu as pltpu
from jax.experimental.pallas import tpu_sc as plsc

N = 10000
E = 320000
D = 128
DW = 8
NC = 2
NS = 16
NW = NC * NS
EPW = E // NW
B = 80
NBLK = EPW // B
G = B // 16


def _sc_mesh():
    return plsc.VectorSubcoreMesh(core_axis_name="c", subcore_axis_name="s")


@functools.partial(
    pl.kernel,
    out_type=(jax.ShapeDtypeStruct((NC * N, D), jnp.float32),
              jax.ShapeDtypeStruct((NC * N, DW), jnp.float32)),
    mesh=_sc_mesh(),
    compiler_params=pltpu.CompilerParams(needs_layout_passes=False,
                                         use_tc_tiling_on_sc=False),
    scratch_types=[
        pltpu.VMEM((D,), jnp.float32),
        pltpu.VMEM((N,), jnp.float32),
        pltpu.VMEM((B,), jnp.int32),
        pltpu.VMEM((B,), jnp.int32),
        pltpu.VMEM((B,), jnp.int32),
        pltpu.VMEM((B,), jnp.int32),
        pltpu.VMEM((B, D), jnp.float32),
        pltpu.VMEM((B, D), jnp.float32),
        pltpu.VMEM((B, D), jnp.float32),
        pltpu.VMEM((B, DW), jnp.float32),
        pltpu.VMEM_SHARED((N, D), jnp.float32),
        pltpu.VMEM_SHARED((N, DW), jnp.float32),
        pltpu.SemaphoreType.DMA,
        pltpu.SemaphoreType.DMA,
        pltpu.SemaphoreType.DMA,
        pltpu.SemaphoreType.DMA,
        pltpu.SemaphoreType.DMA,
        pltpu.SemaphoreType.DMA,
    ],
)
def _sc_main(rel_hbm, src_hbm, dst_hbm, pat_hbm, w1_hbm, b_hbm,
             outh_hbm, outd_hbm,
             w1_v, btab_v, src_v, dst_v, dstA, dstB, relrows_v, pat_v,
             contrib_v, pstg_v, hacc_sh, den_sh,
             gsem, psem, shA, sdA, shB, sdB):
    cid = lax.axis_index("c")
    sid = lax.axis_index("s")
    wid = sid * NC + cid

    pltpu.sync_copy(w1_hbm, w1_v)
    pltpu.sync_copy(b_hbm, btab_v)

    zero16 = jnp.zeros((16,), jnp.float32)
    lane = lax.broadcasted_iota(jnp.int32, (16,), 0)

    def _zrow(r, carry):
        for k in range(D // 16):
            contrib_v[r, pl.ds(k * 16, 16)] = zero16
        return carry

    lax.fori_loop(0, B, _zrow, 0)
    for g in range(G):
        for c in range(DW):
            plsc.store_scatter(pstg_v, [g * 16 + lane,
                                        jnp.full((16,), c, jnp.int32)],
                               zero16)
    nchunk = N // B
    for i in range((nchunk + NS - 1) // NS):
        t = i * NS + sid

        @pl.when(t < nchunk)
        def _():
            pltpu.sync_copy(contrib_v, hacc_sh.at[pl.ds(t * B, B)])
            pltpu.sync_copy(pstg_v, den_sh.at[pl.ds(t * B, B)])

    plsc.subcore_barrier()

    w1c = [w1_v[pl.ds(k * 16, 16)] for k in range(D // 16)]
    zcol = jnp.zeros((16,), jnp.int32)

    def _compute():
        def _group(g, gcarry):
            dst16 = dst_v[pl.ds(g * 16, 16)]
            b16 = plsc.load_gather(btab_v, [dst16])
            pvec = zero16
            for j in range(16):
                e = g * 16 + j
                acc = zero16
                zs = []
                for k in range(D // 16):
                    r = relrows_v[e, pl.ds(k * 16, 16)]
                    pt = pat_v[e, pl.ds(k * 16, 16)]
                    z = r * pt
                    zs.append(z)
                    acc = acc + z * w1c[k]
                acc = acc + jnp.where(lane == j, b16, zero16)
                s = jnp.sum(acc)
                sv = lax.broadcast_in_dim(s, (16,), ())
                sv = jnp.where(sv >= 0, sv, sv * jnp.float32(0.01))
                p = jnp.exp(sv)
                for k in range(D // 16):
                    contrib_v[e, pl.ds(k * 16, 16)] = zs[k] * p
                pvec = jnp.where(lane == j, p, pvec)
            plsc.store_scatter(pstg_v, [g * 16 + lane, zcol], pvec)
            return gcarry

        lax.fori_loop(0, G, _group, 0)

    def _wait_scatter(dstv, sh, sd):
        pltpu.make_async_copy(contrib_v, hacc_sh.at[dstv], sh).wait()
        pltpu.make_async_copy(pstg_v, den_sh.at[dstv], sd).wait()

    def _block(blk, carry):
        base = wid * EPW + blk * B
        even = (blk % 2) == 0
        pltpu.sync_copy(src_hbm.at[pl.ds(base, B)], src_v)
        pltpu.sync_copy(dst_hbm.at[pl.ds(base, B)], dst_v)
        pltpu.async_copy(rel_hbm.at[src_v], relrows_v, gsem)
        pltpu.async_copy(pat_hbm.at[pl.ds(base, B)], pat_v, psem)

        @pl.when(even)
        def _():
            pltpu.sync_copy(dst_hbm.at[pl.ds(base, B)], dstA)

            @pl.when(blk >= 1)
            def _():
                _wait_scatter(dstB, shB, sdB)

        @pl.when(jnp.logical_not(even))
        def _():
            pltpu.sync_copy(dst_hbm.at[pl.ds(base, B)], dstB)
            _wait_scatter(dstA, shA, sdA)

        pltpu.make_async_copy(rel_hbm.at[src_v], relrows_v, gsem).wait()
        pltpu.make_async_copy(pat_hbm.at[pl.ds(base, B)], pat_v, psem).wait()
        _compute()

        @pl.when(even)
        def _():
            pltpu.async_copy(contrib_v, hacc_sh.at[dstA], shA, add=True)
            pltpu.async_copy(pstg_v, den_sh.at[dstA], sdA, add=True)

        @pl.when(jnp.logical_not(even))
        def _():
            pltpu.async_copy(contrib_v, hacc_sh.at[dstB], shB, add=True)
            pltpu.async_copy(pstg_v, den_sh.at[dstB], sdB, add=True)

        return carry

    lax.fori_loop(0, NBLK, _block, 0)
    _wait_scatter(dstA, shA, sdA)

    plsc.subcore_barrier()
    for i in range((nchunk + NS - 1) // NS):
        t = i * NS + sid

        @pl.when(t < nchunk)
        def _():
            pltpu.sync_copy(hacc_sh.at[pl.ds(t * B, B)],
                            outh_hbm.at[pl.ds(cid * N + t * B, B)])
            pltpu.sync_copy(den_sh.at[pl.ds(t * B, B)],
                            outd_hbm.at[pl.ds(cid * N + t * B, B)])


def _pre_body(rel_ref, w2_ref, b_ref):
    b_ref[...] = jnp.sum(rel_ref[...] * w2_ref[...], axis=1, keepdims=True)


def _post_body(acc_ref, den_ref, o_ref):
    hs = acc_ref[0] + acc_ref[1]
    den = den_ref[0, :, 0:1] + den_ref[1, :, 0:1]
    o_ref[...] = jnp.where(den > 0, hs / den, jnp.float32(0.0))


@jax.jit
def kernel(rel, edge_index, pattern, W_attn):
    w1 = W_attn[0, :D]
    w2 = W_attn[0, D:].reshape(1, D)
    src = edge_index[0]
    dst = edge_index[1]

    b = pl.pallas_call(
        _pre_body,
        out_shape=jax.ShapeDtypeStruct((N, 1), jnp.float32),
    )(rel, w2)[:, 0]

    acc, den = _sc_main(rel, src, dst, pattern, w1, b)
    acc = acc.reshape(NC, N, D)
    den = den.reshape(NC, N, DW)

    h = pl.pallas_call(
        _post_body,
        out_shape=jax.ShapeDtypeStruct((N, D), jnp.float32),
    )(acc, den)
    return h
